# baseline (speedup 1.0000x reference)
import functools

import jax
import jax.numpy as jnp
from jax import lax
from jax.experimental import pallas as pl
from jax.experimental.pallas import tpu as pltpu
from jax.experimental.pallas import tpu_sc as plsc

FEAT = 128
TAB = 2 * FEAT
PKW = TAB // 2
NN = 10000
NE = 320000
EPS = 1e-8
THRESH = 0.1
DELTA = 0.0045

NC, NS, L = 2, 16, 16
NW = NC * NS
EPW = NE // NW
EC = 80
NCHUNK = EPW // EC
NG = EC // L


def _tables_body(l_ref, r_ref, w0_ref, w1_ref, a_ref, b_ref, a16_ref, b16_ref):
    w0 = w0_ref[...]
    w1 = w1_ref[...]
    for x_ref, o_ref, o16_ref in ((l_ref, a_ref, a16_ref),
                                  (r_ref, b_ref, b16_ref)):
        x = x_ref[...]
        for h, w in enumerate((w0, w1)):
            wx = x * w
            n = jnp.sqrt(jnp.sum(wx * wx, axis=1, keepdims=True))
            t = wx / jnp.maximum(n, EPS)
            o_ref[:, h * FEAT:(h + 1) * FEAT] = t
            o16_ref[:, h * FEAT:(h + 1) * FEAT] = t.astype(jnp.bfloat16)


def _build_tables(left, right, w0, w1):
    blk = 1000
    grid = NN // blk
    return pl.pallas_call(
        _tables_body,
        grid=(grid,),
        in_specs=[
            pl.BlockSpec((blk, FEAT), lambda i: (i, 0)),
            pl.BlockSpec((blk, FEAT), lambda i: (i, 0)),
            pl.BlockSpec((1, FEAT), lambda i: (0, 0)),
            pl.BlockSpec((1, FEAT), lambda i: (0, 0)),
        ],
        out_specs=[
            pl.BlockSpec((blk, TAB), lambda i: (i, 0)),
            pl.BlockSpec((blk, TAB), lambda i: (i, 0)),
            pl.BlockSpec((blk, TAB), lambda i: (i, 0)),
            pl.BlockSpec((blk, TAB), lambda i: (i, 0)),
        ],
        out_shape=[
            jax.ShapeDtypeStruct((NN, TAB), jnp.float32),
            jax.ShapeDtypeStruct((NN, TAB), jnp.float32),
            jax.ShapeDtypeStruct((NN, TAB), jnp.bfloat16),
            jax.ShapeDtypeStruct((NN, TAB), jnp.bfloat16),
        ],
    )(left, right, w0, w1)


_HIMASK = jnp.int32(-65536)


def _expand_mac(wa, wb, acc_lo, acc_hi):
    lo_a = plsc.bitcast(lax.shift_left(wa, 16), jnp.float32)
    hi_a = plsc.bitcast(lax.bitwise_and(wa, _HIMASK), jnp.float32)
    lo_b = plsc.bitcast(lax.shift_left(wb, 16), jnp.float32)
    hi_b = plsc.bitcast(lax.bitwise_and(wb, _HIMASK), jnp.float32)
    return acc_lo + lo_a * lo_b, acc_hi + hi_a * hi_b


def _sc_edges_body(a16_hbm, b16_hbm, a32_hbm, b32_hbm, src_hbm, dst_hbm,
                   out_hbm, src_v, dst_v, a_bufs, b_bufs, ra_buf, rb_buf,
                   ridx_buf, eid_list, p_buf, out_v, sems, rsems):
    wid = lax.axis_index("s") * NC + lax.axis_index("c")
    base = wid * EPW
    pltpu.sync_copy(src_hbm.at[pl.ds(base, EPW)], src_v)
    pltpu.sync_copy(dst_hbm.at[pl.ds(base, EPW)], dst_v)

    def issue(c, slot):
        off = c * EC
        pltpu.async_copy(a16_hbm.at[src_v.at[pl.ds(off, EC)]], a_bufs[slot],
                         sems[2 * slot])
        pltpu.async_copy(b16_hbm.at[dst_v.at[pl.ds(off, EC)]], b_bufs[slot],
                         sems[2 * slot + 1])

    def wait(slot):
        pltpu.make_async_copy(a16_hbm.at[pl.ds(0, EC)], a_bufs[slot],
                              sems[2 * slot]).wait()
        pltpu.make_async_copy(b16_hbm.at[pl.ds(0, EC)], b_bufs[slot],
                              sems[2 * slot + 1]).wait()

    rows_t = lax.iota(jnp.int32, L)

    def compute(c, slot, cnt):
        a_buf = a_bufs[slot]
        b_buf = b_bufs[slot]

        def group_body(g, cnt):
            e0 = g * L
            for e in range(L):
                r = e0 + e
                al0 = jnp.zeros((L,), jnp.float32)
                ah0 = jnp.zeros((L,), jnp.float32)
                al1 = jnp.zeros((L,), jnp.float32)
                ah1 = jnp.zeros((L,), jnp.float32)
                for k in range(PKW // L):
                    wa = a_buf[r, pl.ds(k * L, L)]
                    wb = b_buf[r, pl.ds(k * L, L)]
                    if k % 2 == 0:
                        al0, ah0 = _expand_mac(wa, wb, al0, ah0)
                    else:
                        al1, ah1 = _expand_mac(wa, wb, al1, ah1)
                p_buf[e, pl.ds(0, L)] = (al0 + al1) + (ah0 + ah1)
            sacc = [jnp.zeros((L,), jnp.float32) for _ in range(4)]
            for k in range(L):
                colk = jnp.full((L,), k, jnp.int32)
                sacc[k % 4] = sacc[k % 4] + plsc.load_gather(p_buf, [rows_t, colk])
            s = ((sacc[0] + sacc[1]) + (sacc[2] + sacc[3])) * jnp.float32(0.5)
            res = jnp.where(s < THRESH, jnp.float32(0.0), s)
            out_v[pl.ds(c * EC + e0, L)] = res
            flag = jnp.abs(s - THRESH) < DELTA
            eid_vec = (c * EC + e0) + rows_t
            plsc.store_compressed(eid_list.at[pl.ds(cnt, L)], eid_vec, mask=flag)
            return cnt + jnp.sum(flag.astype(jnp.int32))

        return lax.fori_loop(0, NG, group_body, cnt, unroll=False)

    issue(0, 0)

    def pair_body(p, cnt):
        c0 = 2 * p
        issue(c0 + 1, 1)
        wait(0)
        cnt = compute(c0, 0, cnt)
        issue(c0 + 2, 0)
        wait(1)
        cnt = compute(c0 + 1, 1, cnt)
        return cnt

    cnt = lax.fori_loop(0, (NCHUNK - 1) // 2, pair_body, jnp.int32(0),
                        unroll=False)
    wait(0)
    cnt = compute(NCHUNK - 1, 0, cnt)

    def rescue_cond(g16):
        return g16 * L < cnt

    def rescue_body(g16):
        lanes = g16 * L + rows_t
        lmask = lanes < cnt
        eidv = plsc.load_gather(eid_list, [jnp.where(lmask, lanes, 0)])
        eidv = jnp.where(lmask, eidv, 0)
        srcv = plsc.load_gather(src_v, [eidv])
        dstv = plsc.load_gather(dst_v, [eidv])
        ridx_buf[pl.ds(0, L)] = srcv
        ridx_buf[pl.ds(L, L)] = dstv
        ca = pltpu.async_copy(a32_hbm.at[ridx_buf.at[pl.ds(0, L)]], ra_buf,
                              rsems[0])
        cb = pltpu.async_copy(b32_hbm.at[ridx_buf.at[pl.ds(L, L)]], rb_buf,
                              rsems[1])
        ca.wait()
        cb.wait()
        for e in range(L):
            accs = [jnp.zeros((L,), jnp.float32) for _ in range(4)]
            for k in range(TAB // L):
                va = ra_buf[e, pl.ds(k * L, L)]
                vb = rb_buf[e, pl.ds(k * L, L)]
                accs[k % 4] = accs[k % 4] + va * vb
            p_buf[e, pl.ds(0, L)] = (accs[0] + accs[1]) + (accs[2] + accs[3])
        sacc = [jnp.zeros((L,), jnp.float32) for _ in range(4)]
        for k in range(L):
            colk = jnp.full((L,), k, jnp.int32)
            sacc[k % 4] = sacc[k % 4] + plsc.load_gather(p_buf, [rows_t, colk])
        s = ((sacc[0] + sacc[1]) + (sacc[2] + sacc[3])) * jnp.float32(0.5)
        res = jnp.where(s < THRESH, jnp.float32(0.0), s)
        plsc.store_scatter(out_v, [eidv], res, mask=lmask)
        return g16 + 1

    lax.while_loop(rescue_cond, rescue_body, jnp.int32(0))

    pltpu.sync_copy(out_v, out_hbm.at[pl.ds(base, EPW)])


@functools.cache
def _sc_edges():
    return pl.kernel(
        _sc_edges_body,
        out_type=jax.ShapeDtypeStruct((NE,), jnp.float32),
        mesh=plsc.VectorSubcoreMesh(core_axis_name="c", subcore_axis_name="s",
                                    num_cores=NC, num_subcores=NS),
        scratch_types=[
            pltpu.VMEM((EPW,), jnp.int32),
            pltpu.VMEM((EPW,), jnp.int32),
            [pltpu.VMEM((EC, PKW), jnp.int32)] * 2,
            [pltpu.VMEM((EC, PKW), jnp.int32)] * 2,
            pltpu.VMEM((L, TAB), jnp.float32),
            pltpu.VMEM((L, TAB), jnp.float32),
            pltpu.VMEM((2 * L,), jnp.int32),
            pltpu.VMEM((EPW + L,), jnp.int32),
            pltpu.VMEM((L, L + 1), jnp.float32),
            pltpu.VMEM((EPW,), jnp.float32),
            [pltpu.SemaphoreType.DMA] * 4,
            [pltpu.SemaphoreType.DMA] * 2,
        ],
        compiler_params=pltpu.CompilerParams(use_tc_tiling_on_sc=False,
                                             needs_layout_passes=False,
                                             disable_bounds_checks=True),
    )


def kernel(left_features, right_features, edge_index, W0, W1):
    a32, b32, a16, b16 = _build_tables(left_features, right_features, W0, W1)
    a16p = lax.bitcast_convert_type(a16.reshape(NN, PKW, 2), jnp.int32)
    b16p = lax.bitcast_convert_type(b16.reshape(NN, PKW, 2), jnp.int32)
    src = edge_index[0]
    dst = edge_index[1]
    return _sc_edges()(a16p, b16p, a32, b32, src, dst)


# speedup vs baseline: 1.3979x; 1.3979x over previous
import functools

import jax
import jax.numpy as jnp
from jax import lax
from jax.experimental import pallas as pl
from jax.experimental.pallas import tpu as pltpu
from jax.experimental.pallas import tpu_sc as plsc

FEAT = 128
TAB = 2 * FEAT
NN = 10000
NE = 320000
EPS = 1e-8
THRESH = 0.1

NC, NS, L = 2, 16, 16
NW = NC * NS
EPW = NE // NW
EC = 80
NCHUNK = EPW // EC


def _tables_body(l_ref, r_ref, w0_ref, w1_ref, a_ref, b_ref):
    w0 = w0_ref[...]
    w1 = w1_ref[...]
    for x_ref, o_ref in ((l_ref, a_ref), (r_ref, b_ref)):
        x = x_ref[...]
        for h, w in enumerate((w0, w1)):
            wx = x * w
            n = jnp.sqrt(jnp.sum(wx * wx, axis=1, keepdims=True))
            o_ref[:, h * FEAT:(h + 1) * FEAT] = wx / jnp.maximum(n, EPS)


def _build_tables(left, right, w0, w1):
    blk = 1000
    grid = NN // blk
    return pl.pallas_call(
        _tables_body,
        grid=(grid,),
        in_specs=[
            pl.BlockSpec((blk, FEAT), lambda i: (i, 0)),
            pl.BlockSpec((blk, FEAT), lambda i: (i, 0)),
            pl.BlockSpec((1, FEAT), lambda i: (0, 0)),
            pl.BlockSpec((1, FEAT), lambda i: (0, 0)),
        ],
        out_specs=[
            pl.BlockSpec((blk, TAB), lambda i: (i, 0)),
            pl.BlockSpec((blk, TAB), lambda i: (i, 0)),
        ],
        out_shape=[
            jax.ShapeDtypeStruct((NN, TAB), jnp.float32),
            jax.ShapeDtypeStruct((NN, TAB), jnp.float32),
        ],
    )(left, right, w0, w1)


def _sc_edges_body(a_hbm, b_hbm, src_hbm, dst_hbm, out_hbm,
                   src_v, dst_v, a_bufs, b_bufs, p_buf, out_v, sems):
    wid = lax.axis_index("s") * NC + lax.axis_index("c")
    base = wid * EPW
    pltpu.sync_copy(src_hbm.at[pl.ds(base, EPW)], src_v)
    pltpu.sync_copy(dst_hbm.at[pl.ds(base, EPW)], dst_v)

    def issue(c, slot):
        off = c * EC
        pltpu.async_copy(a_hbm.at[src_v.at[pl.ds(off, EC)]], a_bufs[slot],
                         sems[2 * slot])
        pltpu.async_copy(b_hbm.at[dst_v.at[pl.ds(off, EC)]], b_bufs[slot],
                         sems[2 * slot + 1])

    def wait(slot):
        pltpu.make_async_copy(a_hbm.at[pl.ds(0, EC)], a_bufs[slot],
                              sems[2 * slot]).wait()
        pltpu.make_async_copy(b_hbm.at[pl.ds(0, EC)], b_bufs[slot],
                              sems[2 * slot + 1]).wait()

    def compute(c, slot):
        a_buf = a_bufs[slot]
        b_buf = b_bufs[slot]

        def group_body(g, _):
            e0 = g * L
            rows_t = lax.iota(jnp.int32, L)
            for e in range(L):
                r = e0 + e
                accs = [jnp.zeros((L,), jnp.float32) for _ in range(4)]
                for k in range(TAB // L):
                    va = a_buf[r, pl.ds(k * L, L)]
                    vb = b_buf[r, pl.ds(k * L, L)]
                    accs[k % 4] = accs[k % 4] + va * vb
                acc = (accs[0] + accs[1]) + (accs[2] + accs[3])
                p_buf[e, pl.ds(0, L)] = acc
            sacc = [jnp.zeros((L,), jnp.float32) for _ in range(4)]
            for k in range(L):
                colk = jnp.full((L,), k, jnp.int32)
                sacc[k % 4] = sacc[k % 4] + plsc.load_gather(p_buf, [rows_t, colk])
            s = ((sacc[0] + sacc[1]) + (sacc[2] + sacc[3])) * jnp.float32(0.5)
            res = jnp.where(s < THRESH, jnp.float32(0.0), s)
            out_v[pl.ds(c * EC + e0, L)] = res
            return 0

        lax.fori_loop(0, EC // L, group_body, 0, unroll=False)

    issue(0, 0)

    def pair_body(p, _):
        c0 = 2 * p
        issue(c0 + 1, 1)
        wait(0)
        compute(c0, 0)
        issue(c0 + 2, 0)
        wait(1)
        compute(c0 + 1, 1)
        return 0

    lax.fori_loop(0, (NCHUNK - 1) // 2, pair_body, 0, unroll=False)
    wait(0)
    compute(NCHUNK - 1, 0)

    pltpu.sync_copy(out_v, out_hbm.at[pl.ds(base, EPW)])


@functools.cache
def _sc_edges():
    return pl.kernel(
        _sc_edges_body,
        out_type=jax.ShapeDtypeStruct((NE,), jnp.float32),
        mesh=plsc.VectorSubcoreMesh(core_axis_name="c", subcore_axis_name="s",
                                    num_cores=NC, num_subcores=NS),
        scratch_types=[
            pltpu.VMEM((EPW,), jnp.int32),
            pltpu.VMEM((EPW,), jnp.int32),
            [pltpu.VMEM((EC, TAB), jnp.float32)] * 2,
            [pltpu.VMEM((EC, TAB), jnp.float32)] * 2,
            pltpu.VMEM((L, L + 1), jnp.float32),
            pltpu.VMEM((EPW,), jnp.float32),
            [pltpu.SemaphoreType.DMA] * 4,
        ],
        compiler_params=pltpu.CompilerParams(use_tc_tiling_on_sc=False,
                                             needs_layout_passes=False,
                                             disable_bounds_checks=True),
    )


def kernel(left_features, right_features, edge_index, W0, W1):
    a_tab, b_tab = _build_tables(left_features, right_features, W0, W1)
    src = edge_index[0]
    dst = edge_index[1]
    return _sc_edges()(a_tab, b_tab, src, dst)


# speedup vs baseline: 1.3995x; 1.0012x over previous
import functools

import jax
import jax.numpy as jnp
from jax import lax
from jax.experimental import pallas as pl
from jax.experimental.pallas import tpu as pltpu
from jax.experimental.pallas import tpu_sc as plsc

FEAT = 128
TAB = 2 * FEAT
NN = 10000
NE = 320000
EPS = 1e-8
THRESH = 0.1

NC, NS, L = 2, 16, 16
NW = NC * NS
EPW = NE // NW
EC = 80
NCHUNK = EPW // EC


def _tables_body(l_ref, r_ref, w0_ref, w1_ref, a_ref, b_ref):
    w0 = w0_ref[...]
    w1 = w1_ref[...]
    for x_ref, o_ref in ((l_ref, a_ref), (r_ref, b_ref)):
        x = x_ref[...]
        for h, w in enumerate((w0, w1)):
            wx = x * w
            n = jnp.sqrt(jnp.sum(wx * wx, axis=1, keepdims=True))
            o_ref[:, h * FEAT:(h + 1) * FEAT] = wx / jnp.maximum(n, EPS)


def _build_tables(left, right, w0, w1):
    blk = 1000
    grid = NN // blk
    return pl.pallas_call(
        _tables_body,
        grid=(grid,),
        in_specs=[
            pl.BlockSpec((blk, FEAT), lambda i: (i, 0)),
            pl.BlockSpec((blk, FEAT), lambda i: (i, 0)),
            pl.BlockSpec((1, FEAT), lambda i: (0, 0)),
            pl.BlockSpec((1, FEAT), lambda i: (0, 0)),
        ],
        out_specs=[
            pl.BlockSpec((blk, TAB), lambda i: (i, 0)),
            pl.BlockSpec((blk, TAB), lambda i: (i, 0)),
        ],
        out_shape=[
            jax.ShapeDtypeStruct((NN, TAB), jnp.float32),
            jax.ShapeDtypeStruct((NN, TAB), jnp.float32),
        ],
    )(left, right, w0, w1)


def _sc_edges_body(a_hbm, b_hbm, ei_hbm, out_hbm,
                   src_v, dst_v, a_bufs, b_bufs, p_buf, out_v, sems):
    wid = lax.axis_index("s") * NC + lax.axis_index("c")
    base = wid * EPW
    pltpu.sync_copy(ei_hbm.at[0, pl.ds(base, EPW)], src_v)
    pltpu.sync_copy(ei_hbm.at[1, pl.ds(base, EPW)], dst_v)

    def issue(c, slot):
        off = c * EC
        pltpu.async_copy(a_hbm.at[src_v.at[pl.ds(off, EC)]], a_bufs[slot],
                         sems[2 * slot])
        pltpu.async_copy(b_hbm.at[dst_v.at[pl.ds(off, EC)]], b_bufs[slot],
                         sems[2 * slot + 1])

    def wait(slot):
        pltpu.make_async_copy(a_hbm.at[pl.ds(0, EC)], a_bufs[slot],
                              sems[2 * slot]).wait()
        pltpu.make_async_copy(b_hbm.at[pl.ds(0, EC)], b_bufs[slot],
                              sems[2 * slot + 1]).wait()

    def compute(c, slot):
        a_buf = a_bufs[slot]
        b_buf = b_bufs[slot]

        def group_body(g, _):
            e0 = g * L
            rows_t = lax.iota(jnp.int32, L)
            for e in range(L):
                r = e0 + e
                accs = [jnp.zeros((L,), jnp.float32) for _ in range(4)]
                for k in range(TAB // L):
                    va = a_buf[r, pl.ds(k * L, L)]
                    vb = b_buf[r, pl.ds(k * L, L)]
                    accs[k % 4] = accs[k % 4] + va * vb
                acc = (accs[0] + accs[1]) + (accs[2] + accs[3])
                p_buf[e, pl.ds(0, L)] = acc
            sacc = [jnp.zeros((L,), jnp.float32) for _ in range(4)]
            for k in range(L):
                colk = jnp.full((L,), k, jnp.int32)
                sacc[k % 4] = sacc[k % 4] + plsc.load_gather(p_buf, [rows_t, colk])
            s = ((sacc[0] + sacc[1]) + (sacc[2] + sacc[3])) * jnp.float32(0.5)
            res = jnp.where(s < THRESH, jnp.float32(0.0), s)
            out_v[pl.ds(c * EC + e0, L)] = res
            return 0

        lax.fori_loop(0, EC // L, group_body, 0, unroll=False)

    issue(0, 0)

    def pair_body(p, _):
        c0 = 2 * p
        issue(c0 + 1, 1)
        wait(0)
        compute(c0, 0)
        issue(c0 + 2, 0)
        wait(1)
        compute(c0 + 1, 1)
        return 0

    lax.fori_loop(0, (NCHUNK - 1) // 2, pair_body, 0, unroll=False)
    wait(0)
    compute(NCHUNK - 1, 0)

    pltpu.sync_copy(out_v, out_hbm.at[pl.ds(base, EPW)])


@functools.cache
def _sc_edges():
    return pl.kernel(
        _sc_edges_body,
        out_type=jax.ShapeDtypeStruct((NE,), jnp.float32),
        mesh=plsc.VectorSubcoreMesh(core_axis_name="c", subcore_axis_name="s",
                                    num_cores=NC, num_subcores=NS),
        scratch_types=[
            pltpu.VMEM((EPW,), jnp.int32),
            pltpu.VMEM((EPW,), jnp.int32),
            [pltpu.VMEM((EC, TAB), jnp.float32)] * 2,
            [pltpu.VMEM((EC, TAB), jnp.float32)] * 2,
            pltpu.VMEM((L, L + 1), jnp.float32),
            pltpu.VMEM((EPW,), jnp.float32),
            [pltpu.SemaphoreType.DMA] * 4,
        ],
        compiler_params=pltpu.CompilerParams(use_tc_tiling_on_sc=False,
                                             needs_layout_passes=False,
                                             disable_bounds_checks=True),
    )


def kernel(left_features, right_features, edge_index, W0, W1):
    a_tab, b_tab = _build_tables(left_features, right_features, W0, W1)
    return _sc_edges()(a_tab, b_tab, edge_index)


# speedup vs baseline: 1.4158x; 1.0116x over previous
import functools

import jax
import jax.numpy as jnp
from jax import lax
from jax.experimental import pallas as pl
from jax.experimental.pallas import tpu as pltpu
from jax.experimental.pallas import tpu_sc as plsc

FEAT = 128
TAB = 2 * FEAT
NN = 10000
NE = 320000
EPS = 1e-8
THRESH = 0.1

NC, NS, L = 2, 16, 16
NW = NC * NS
EPW = NE // NW
EC = 64
NFULL = EPW // EC
TAILE = EPW - NFULL * EC
CIW = 2 * EC


def _tables_body(l_ref, r_ref, w0_ref, w1_ref, t_ref):
    s = pl.program_id(0)
    w0 = w0_ref[...]
    w1 = w1_ref[...]
    x = jnp.where(s == 0, l_ref[...], r_ref[...])
    for h, w in enumerate((w0, w1)):
        wx = x * w
        n = jnp.sqrt(jnp.sum(wx * wx, axis=1, keepdims=True))
        t_ref[:, h * FEAT:(h + 1) * FEAT] = wx / jnp.maximum(n, EPS)


def _build_tables(left, right, w0, w1):
    blk = 1000
    grid = NN // blk
    return pl.pallas_call(
        _tables_body,
        grid=(2, grid),
        in_specs=[
            pl.BlockSpec((blk, FEAT), lambda s, i: (i, 0)),
            pl.BlockSpec((blk, FEAT), lambda s, i: (i, 0)),
            pl.BlockSpec((1, FEAT), lambda s, i: (0, 0)),
            pl.BlockSpec((1, FEAT), lambda s, i: (0, 0)),
        ],
        out_specs=pl.BlockSpec((blk, TAB), lambda s, i: (s * grid + i, 0)),
        out_shape=jax.ShapeDtypeStruct((2 * NN, TAB), jnp.float32),
    )(left, right, w0, w1)


def _sc_edges_body(t_hbm, ei_hbm, out_hbm,
                   src_v, dst_v, cidx_v, bufs, tail_buf, p_buf, out_v,
                   sems, tsem):
    wid = lax.axis_index("s") * NC + lax.axis_index("c")
    base = wid * EPW
    pltpu.sync_copy(ei_hbm.at[0, pl.ds(base, EPW)], src_v)
    pltpu.sync_copy(ei_hbm.at[1, pl.ds(base, EPW)], dst_v)

    def cidx_body(c, _):
        for j in range(EC // L):
            cidx_v[pl.ds(c * CIW + j * L, L)] = src_v[pl.ds(c * EC + j * L, L)]
            cidx_v[pl.ds(c * CIW + EC + j * L, L)] = (
                dst_v[pl.ds(c * EC + j * L, L)] + NN)
        return 0

    lax.fori_loop(0, NFULL, cidx_body, 0, unroll=False)
    cidx_v[pl.ds(NFULL * CIW, L)] = src_v[pl.ds(NFULL * EC, L)]
    cidx_v[pl.ds(NFULL * CIW + L, L)] = dst_v[pl.ds(NFULL * EC, L)] + NN

    def issue(c, slot):
        pltpu.async_copy(t_hbm.at[cidx_v.at[pl.ds(c * CIW, CIW)]],
                         bufs[slot], sems[slot])

    def wait(slot):
        pltpu.make_async_copy(t_hbm.at[pl.ds(0, CIW)], bufs[slot],
                              sems[slot]).wait()

    rows_t = lax.iota(jnp.int32, L)

    def dot_groups(buf, b_off, ngroups, out_off):
        def group_body(g, _):
            e0 = g * L
            for e in range(L):
                ra = e0 + e
                rb = b_off + e0 + e
                accs = [jnp.zeros((L,), jnp.float32) for _ in range(4)]
                for k in range(TAB // L):
                    va = buf[ra, pl.ds(k * L, L)]
                    vb = buf[rb, pl.ds(k * L, L)]
                    accs[k % 4] = accs[k % 4] + va * vb
                acc = (accs[0] + accs[1]) + (accs[2] + accs[3])
                p_buf[e, pl.ds(0, L)] = acc
            sacc = [jnp.zeros((L,), jnp.float32) for _ in range(4)]
            for k in range(L):
                colk = jnp.full((L,), k, jnp.int32)
                sacc[k % 4] = sacc[k % 4] + plsc.load_gather(p_buf, [rows_t, colk])
            s = ((sacc[0] + sacc[1]) + (sacc[2] + sacc[3])) * jnp.float32(0.5)
            res = jnp.where(s < THRESH, jnp.float32(0.0), s)
            out_v[pl.ds(out_off + e0, L)] = res
            return 0

        lax.fori_loop(0, ngroups, group_body, 0, unroll=False)

    def compute(c, slot):
        dot_groups(bufs[slot], EC, EC // L, c * EC)

    pltpu.async_copy(t_hbm.at[cidx_v.at[pl.ds(NFULL * CIW, 2 * TAILE)]],
                     tail_buf, tsem)
    issue(0, 0)

    def pair_body(p, _):
        c0 = 2 * p
        issue(c0 + 1, 1)
        wait(0)
        compute(c0, 0)
        issue(c0 + 2, 0)
        wait(1)
        compute(c0 + 1, 1)
        return 0

    lax.fori_loop(0, NFULL // 2 - 1, pair_body, 0, unroll=False)
    issue(NFULL - 1, 1)
    wait(0)
    compute(NFULL - 2, 0)
    wait(1)
    compute(NFULL - 1, 1)

    pltpu.make_async_copy(t_hbm.at[pl.ds(0, 2 * TAILE)], tail_buf, tsem).wait()
    dot_groups(tail_buf, TAILE, 1, NFULL * EC)

    pltpu.sync_copy(out_v, out_hbm.at[pl.ds(base, EPW)])


@functools.cache
def _sc_edges():
    return pl.kernel(
        _sc_edges_body,
        out_type=jax.ShapeDtypeStruct((NE,), jnp.float32),
        mesh=plsc.VectorSubcoreMesh(core_axis_name="c", subcore_axis_name="s",
                                    num_cores=NC, num_subcores=NS),
        scratch_types=[
            pltpu.VMEM((EPW,), jnp.int32),
            pltpu.VMEM((EPW,), jnp.int32),
            pltpu.VMEM((2 * EPW,), jnp.int32),
            [pltpu.VMEM((CIW, TAB), jnp.float32)] * 2,
            pltpu.VMEM((2 * TAILE, TAB), jnp.float32),
            pltpu.VMEM((L, L + 1), jnp.float32),
            pltpu.VMEM((EPW,), jnp.float32),
            [pltpu.SemaphoreType.DMA] * 2,
            pltpu.SemaphoreType.DMA,
        ],
        compiler_params=pltpu.CompilerParams(use_tc_tiling_on_sc=False,
                                             needs_layout_passes=False,
                                             disable_bounds_checks=True),
    )


def kernel(left_features, right_features, edge_index, W0, W1):
    t_tab = _build_tables(left_features, right_features, W0, W1)
    return _sc_edges()(t_tab, edge_index)
